# Initial kernel scaffold; baseline (speedup 1.0000x reference)
import jax
import jax.numpy as jnp
from jax.experimental import pallas as pl


def kernel(x, vertex, edges, degE, degV, W0, b0, W1, W2, Wout, bout):
    raise NotImplementedError("write your pallas kernel here")


# speedup vs baseline: 3.5747x; 3.5747x over previous
import functools
import math

import jax
import jax.numpy as jnp
from jax import lax
from jax.experimental import pallas as pl
from jax.experimental.pallas import tpu as pltpu
from jax.experimental.pallas import tpu_sc as plsc

N = 10000
NP = 10240
NNZ = 320000
NE = 5000
NEP = 5120
NHID = 128
NCLASS = 40

CHUNK = 128
NCHUNKS = NNZ // CHUNK
SC_CHUNKS = NCHUNKS // 2
TILE_CHUNKS = 79
ROW_BLK = 1024

_i32 = jnp.int32
_f32 = jnp.float32


def _chunk_range(c, s):
    m0 = s * TILE_CHUNKS
    cnt = jnp.maximum(jnp.minimum(m0 + TILE_CHUNKS, SC_CHUNKS) - m0, 0)
    return c * SC_CHUNKS + m0, cnt


def _zero_rows(zrow_v, dst_sh, row0, n16):
    def zb(k, _):
        pltpu.sync_copy(zrow_v, dst_sh.at[pl.ds(row0 + k * 16, 16)])
        return 0

    lax.fori_loop(0, n16, zb, 0)


def _init_zrow(zrow_v):
    def zrow_body(i, _):
        for j in range(8):
            zrow_v[i, pl.ds(j * 16, 16)] = jnp.zeros((16,), _f32)
        return 0

    lax.fori_loop(0, 16, zrow_body, 0)


def _sc_edge_body(h_hbm, vtx_hbm, edg_hbm, out_hbm, xe_sh, rows_v, ev_v, vv_v,
                  zrow_v):
    c = lax.axis_index("c")
    s = lax.axis_index("s")
    _init_zrow(zrow_v)
    _zero_rows(zrow_v, xe_sh, s * 320, 20)
    plsc.subcore_barrier()

    m0, cnt = _chunk_range(c, s)

    def stage(ci, _):
        p0 = (m0 + ci) * CHUNK
        pltpu.sync_copy(vtx_hbm.at[pl.ds(p0, CHUNK)], vv_v)
        pltpu.sync_copy(edg_hbm.at[pl.ds(p0, CHUNK)], ev_v)
        pltpu.sync_copy(h_hbm.at[vv_v], rows_v)
        pltpu.sync_copy(rows_v, xe_sh.at[ev_v], add=True)
        return 0

    lax.fori_loop(0, cnt, stage, 0)
    plsc.subcore_barrier()
    pltpu.sync_copy(xe_sh.at[pl.ds(s * 320, 320)], out_hbm.at[c, pl.ds(s * 320, 320)])


_sc_edge = functools.partial(
    pl.kernel,
    out_type=jax.ShapeDtypeStruct((2, NEP, NHID), _f32),
    mesh=plsc.VectorSubcoreMesh(core_axis_name="c", subcore_axis_name="s"),
    scratch_types=[
        pltpu.VMEM_SHARED((NEP, NHID), _f32),
        pltpu.VMEM((CHUNK, NHID), _f32),
        pltpu.VMEM((CHUNK,), _i32),
        pltpu.VMEM((CHUNK,), _i32),
        pltpu.VMEM((16, NHID), _f32),
    ],
)(_sc_edge_body)


def _sc_node_body(xe_hbm, vtx_hbm, edg_hbm, out_hbm, xv_sh, rows_v, ev_v, vv_v,
                  zrow_v):
    c = lax.axis_index("c")
    s = lax.axis_index("s")
    _init_zrow(zrow_v)
    _zero_rows(zrow_v, xv_sh, s * 640, 40)
    plsc.subcore_barrier()

    m0, cnt = _chunk_range(c, s)

    def stage(ci, _):
        p0 = (m0 + ci) * CHUNK
        pltpu.sync_copy(edg_hbm.at[pl.ds(p0, CHUNK)], ev_v)
        pltpu.sync_copy(vtx_hbm.at[pl.ds(p0, CHUNK)], vv_v)
        pltpu.sync_copy(xe_hbm.at[ev_v], rows_v)
        pltpu.sync_copy(rows_v, xv_sh.at[vv_v], add=True)
        return 0

    lax.fori_loop(0, cnt, stage, 0)
    plsc.subcore_barrier()
    pltpu.sync_copy(xv_sh.at[pl.ds(s * 640, 640)], out_hbm.at[c, pl.ds(s * 640, 640)])


_sc_node = functools.partial(
    pl.kernel,
    out_type=jax.ShapeDtypeStruct((2, NP, NHID), _f32),
    mesh=plsc.VectorSubcoreMesh(core_axis_name="c", subcore_axis_name="s"),
    scratch_types=[
        pltpu.VMEM_SHARED((NP, NHID), _f32),
        pltpu.VMEM((CHUNK, NHID), _f32),
        pltpu.VMEM((CHUNK,), _i32),
        pltpu.VMEM((CHUNK,), _i32),
        pltpu.VMEM((16, NHID), _f32),
    ],
)(_sc_node_body)


def _k0_body(x_ref, w_ref, b_ref, o_ref):
    acc = jnp.dot(x_ref[...], w_ref[...], preferred_element_type=_f32)
    o_ref[...] = jnp.maximum(acc + b_ref[...], 0.0)


def _tc_input_layer(x, w0, b0):
    return pl.pallas_call(
        _k0_body,
        grid=(NP // ROW_BLK,),
        in_specs=[
            pl.BlockSpec((ROW_BLK, NHID), lambda i: (i, 0)),
            pl.BlockSpec((NHID, NHID), lambda i: (0, 0)),
            pl.BlockSpec((1, NHID), lambda i: (0, 0)),
        ],
        out_specs=pl.BlockSpec((ROW_BLK, NHID), lambda i: (i, 0)),
        out_shape=jax.ShapeDtypeStruct((NP, NHID), _f32),
    )(x, w0, b0.reshape(1, NHID))


def _kc_body(p_ref, d_ref, o_ref):
    d = d_ref[...]
    o_ref[...] = (p_ref[0] + p_ref[1]) * (d * d * d)


def _tc_edge_scale(xep, dE):
    return pl.pallas_call(
        _kc_body,
        grid=(NEP // ROW_BLK,),
        in_specs=[
            pl.BlockSpec((2, ROW_BLK, NHID), lambda i: (0, i, 0)),
            pl.BlockSpec((ROW_BLK, 1), lambda i: (i, 0)),
        ],
        out_specs=pl.BlockSpec((ROW_BLK, NHID), lambda i: (i, 0)),
        out_shape=jax.ShapeDtypeStruct((NEP, NHID), _f32),
    )(xep, dE)


def _k1_body(beta, xvp_ref, dv_ref, h0_ref, w_ref, o_ref):
    xv = xvp_ref[0] + xvp_ref[1]
    xi = xv * dv_ref[...] + 0.1 * h0_ref[...]
    acc = jnp.dot(xi, w_ref[...], preferred_element_type=_f32)
    o_ref[...] = jnp.maximum((1.0 - beta) * xi + beta * acc, 0.0)


def _tc_gcnii_layer(xvp, dv9, h0, w, beta):
    return pl.pallas_call(
        functools.partial(_k1_body, beta),
        grid=(NP // ROW_BLK,),
        in_specs=[
            pl.BlockSpec((2, ROW_BLK, NHID), lambda i: (0, i, 0)),
            pl.BlockSpec((ROW_BLK, 1), lambda i: (i, 0)),
            pl.BlockSpec((ROW_BLK, NHID), lambda i: (i, 0)),
            pl.BlockSpec((NHID, NHID), lambda i: (0, 0)),
        ],
        out_specs=pl.BlockSpec((ROW_BLK, NHID), lambda i: (i, 0)),
        out_shape=jax.ShapeDtypeStruct((NP, NHID), _f32),
    )(xvp, dv9, h0, w)


def _k2_body(h_ref, w_ref, b_ref, o_ref):
    acc = jnp.dot(h_ref[...], w_ref[...], preferred_element_type=_f32)
    o_ref[...] = acc + b_ref[...]


def _tc_output_layer(h, wout, bout):
    return pl.pallas_call(
        _k2_body,
        grid=(NP // ROW_BLK,),
        in_specs=[
            pl.BlockSpec((ROW_BLK, NHID), lambda i: (i, 0)),
            pl.BlockSpec((NHID, NCLASS), lambda i: (0, 0)),
            pl.BlockSpec((1, NCLASS), lambda i: (0, 0)),
        ],
        out_specs=pl.BlockSpec((ROW_BLK, NCLASS), lambda i: (i, 0)),
        out_shape=jax.ShapeDtypeStruct((NP, NCLASS), _f32),
    )(h, wout, bout.reshape(1, NCLASS))


def kernel(x, vertex, edges, degE, degV, W0, b0, W1, W2, Wout, bout):
    lamda, alpha = 0.5, 0.1
    vertex = vertex.astype(_i32)
    edges = edges.astype(_i32)

    dE = jnp.pad(degE.reshape(NE, 1), ((0, NEP - NE), (0, 0)))
    dv9 = jnp.pad((1.0 - alpha) * degV, ((0, NP - N), (0, 0)))
    xp = jnp.pad(x, ((0, NP - N), (0, 0)))

    h = _tc_input_layer(xp, W0, b0)
    h0 = h
    for i, w in enumerate([W1, W2]):
        beta = math.log(lamda / (i + 1) + 1)
        xep = _sc_edge(h, vertex, edges)
        xe = _tc_edge_scale(xep, dE)
        xvp = _sc_node(xe, vertex, edges)
        h = _tc_gcnii_layer(xvp, dv9, h0, w, beta)
    return _tc_output_layer(h, Wout, bout)[:N]


# speedup vs baseline: 4.1340x; 1.1564x over previous
import functools
import math

import jax
import jax.numpy as jnp
from jax import lax
from jax.experimental import pallas as pl
from jax.experimental.pallas import tpu as pltpu
from jax.experimental.pallas import tpu_sc as plsc

N = 10000
NP = 10240
NNZ = 320000
NE = 5000
NEP = 5120
NHID = 128
NCLASS = 40

CHUNK = 80
NCH_T = 125
ROW_BLK = 1024

_i32 = jnp.int32
_f32 = jnp.float32


def _zero_rows(zrow_v, dst_sh, row0, n16):
    def zb(k, _):
        pltpu.sync_copy(zrow_v, dst_sh.at[pl.ds(row0 + k * 16, 16)])
        return 0

    lax.fori_loop(0, n16, zb, 0)


def _init_zrow(zrow_v):
    def zrow_body(i, _):
        for j in range(8):
            zrow_v[i, pl.ds(j * 16, 16)] = jnp.zeros((16,), _f32)
        return 0

    lax.fori_loop(0, 16, zrow_body, 0)


def _pipelined_stage(data_hbm, acc_sh, gidx_v, sidx_v, rows, gsem, ssem):
    def start_g(i, b):
        pltpu.async_copy(data_hbm.at[gidx_v.at[pl.ds(i * CHUNK, CHUNK)]],
                         rows[b], gsem[b])

    def wait_g(b):
        pltpu.make_async_copy(data_hbm.at[gidx_v.at[pl.ds(0, CHUNK)]],
                              rows[b], gsem[b]).wait()

    def start_s(i, b):
        pltpu.async_copy(rows[b], acc_sh.at[sidx_v.at[i]], ssem[b], add=True)

    def wait_s(b):
        pltpu.make_async_copy(rows[b], acc_sh.at[sidx_v.at[0]], ssem[b]).wait()

    start_g(0, 0)

    def body(k, _):
        ia = 2 * k
        wait_g(0)

        @pl.when(k > 0)
        def _():
            wait_s(1)

        start_g(ia + 1, 1)
        start_s(ia, 0)
        wait_g(1)
        wait_s(0)
        start_g(ia + 2, 0)
        start_s(ia + 1, 1)
        return 0

    lax.fori_loop(0, (NCH_T - 1) // 2, body, 0)
    wait_g(0)
    wait_s(1)
    start_s(NCH_T - 1, 0)
    wait_s(0)


def _sc_stage_body(rows_total, data_hbm, gidx_hbm, sidx_hbm, out_hbm, acc_sh,
                   gidx_v, sidx_v, rows_a, rows_b, zrow_v, gsem_a, gsem_b,
                   ssem_a, ssem_b):
    c = lax.axis_index("c")
    s = lax.axis_index("s")
    rows_t = rows_total // 16
    _init_zrow(zrow_v)
    _zero_rows(zrow_v, acc_sh, s * rows_t, rows_t // 16)
    pltpu.sync_copy(gidx_hbm.at[c, s], gidx_v)
    pltpu.sync_copy(sidx_hbm.at[c, s], sidx_v)
    plsc.subcore_barrier()

    _pipelined_stage(data_hbm, acc_sh, gidx_v, sidx_v,
                     (rows_a, rows_b), (gsem_a, gsem_b), (ssem_a, ssem_b))

    plsc.subcore_barrier()
    pltpu.sync_copy(acc_sh.at[pl.ds(s * rows_t, rows_t)],
                    out_hbm.at[c, pl.ds(s * rows_t, rows_t)])


def _make_sc_stage(acc_rows):
    return functools.partial(
        pl.kernel,
        out_type=jax.ShapeDtypeStruct((2, acc_rows, NHID), _f32),
        mesh=plsc.VectorSubcoreMesh(core_axis_name="c", subcore_axis_name="s"),
        scratch_types=[
            pltpu.VMEM_SHARED((acc_rows, NHID), _f32),
            pltpu.VMEM((NCH_T * CHUNK,), _i32),
            pltpu.VMEM((NCH_T, CHUNK), _i32),
            pltpu.VMEM((CHUNK, NHID), _f32),
            pltpu.VMEM((CHUNK, NHID), _f32),
            pltpu.VMEM((16, NHID), _f32),
            pltpu.SemaphoreType.DMA,
            pltpu.SemaphoreType.DMA,
            pltpu.SemaphoreType.DMA,
            pltpu.SemaphoreType.DMA,
        ],
    )(functools.partial(_sc_stage_body, acc_rows))


_sc_edge = _make_sc_stage(NEP)
_sc_node = _make_sc_stage(NP)


def _k0_body(x_ref, w_ref, b_ref, o_ref):
    acc = jnp.dot(x_ref[...], w_ref[...], preferred_element_type=_f32)
    o_ref[...] = jnp.maximum(acc + b_ref[...], 0.0)


def _tc_input_layer(x, w0, b0):
    return pl.pallas_call(
        _k0_body,
        grid=(NP // ROW_BLK,),
        in_specs=[
            pl.BlockSpec((ROW_BLK, NHID), lambda i: (i, 0)),
            pl.BlockSpec((NHID, NHID), lambda i: (0, 0)),
            pl.BlockSpec((1, NHID), lambda i: (0, 0)),
        ],
        out_specs=pl.BlockSpec((ROW_BLK, NHID), lambda i: (i, 0)),
        out_shape=jax.ShapeDtypeStruct((NP, NHID), _f32),
    )(x, w0, b0.reshape(1, NHID))


def _kc_body(p_ref, d_ref, o_ref):
    d = d_ref[...]
    o_ref[...] = (p_ref[0] + p_ref[1]) * (d * d * d)


def _tc_edge_scale(xep, dE):
    return pl.pallas_call(
        _kc_body,
        grid=(NEP // ROW_BLK,),
        in_specs=[
            pl.BlockSpec((2, ROW_BLK, NHID), lambda i: (0, i, 0)),
            pl.BlockSpec((ROW_BLK, 1), lambda i: (i, 0)),
        ],
        out_specs=pl.BlockSpec((ROW_BLK, NHID), lambda i: (i, 0)),
        out_shape=jax.ShapeDtypeStruct((NEP, NHID), _f32),
    )(xep, dE)


def _k1_body(beta, xvp_ref, dv_ref, h0_ref, w_ref, o_ref):
    xv = xvp_ref[0] + xvp_ref[1]
    xi = xv * dv_ref[...] + 0.1 * h0_ref[...]
    acc = jnp.dot(xi, w_ref[...], preferred_element_type=_f32)
    o_ref[...] = jnp.maximum((1.0 - beta) * xi + beta * acc, 0.0)


def _tc_gcnii_layer(xvp, dv9, h0, w, beta):
    return pl.pallas_call(
        functools.partial(_k1_body, beta),
        grid=(NP // ROW_BLK,),
        in_specs=[
            pl.BlockSpec((2, ROW_BLK, NHID), lambda i: (0, i, 0)),
            pl.BlockSpec((ROW_BLK, 1), lambda i: (i, 0)),
            pl.BlockSpec((ROW_BLK, NHID), lambda i: (i, 0)),
            pl.BlockSpec((NHID, NHID), lambda i: (0, 0)),
        ],
        out_specs=pl.BlockSpec((ROW_BLK, NHID), lambda i: (i, 0)),
        out_shape=jax.ShapeDtypeStruct((NP, NHID), _f32),
    )(xvp, dv9, h0, w)


def _k1o_body(beta, xvp_ref, dv_ref, h0_ref, w_ref, wo_ref, bo_ref, o_ref):
    xv = xvp_ref[0] + xvp_ref[1]
    xi = xv * dv_ref[...] + 0.1 * h0_ref[...]
    acc = jnp.dot(xi, w_ref[...], preferred_element_type=_f32)
    h = jnp.maximum((1.0 - beta) * xi + beta * acc, 0.0)
    o_ref[...] = jnp.dot(h, wo_ref[...], preferred_element_type=_f32) + bo_ref[...]


def _tc_gcnii_out_layer(xvp, dv9, h0, w, beta, wout, bout):
    return pl.pallas_call(
        functools.partial(_k1o_body, beta),
        grid=(NP // ROW_BLK,),
        in_specs=[
            pl.BlockSpec((2, ROW_BLK, NHID), lambda i: (0, i, 0)),
            pl.BlockSpec((ROW_BLK, 1), lambda i: (i, 0)),
            pl.BlockSpec((ROW_BLK, NHID), lambda i: (i, 0)),
            pl.BlockSpec((NHID, NHID), lambda i: (0, 0)),
            pl.BlockSpec((NHID, NCLASS), lambda i: (0, 0)),
            pl.BlockSpec((1, NCLASS), lambda i: (0, 0)),
        ],
        out_specs=pl.BlockSpec((ROW_BLK, NCLASS), lambda i: (i, 0)),
        out_shape=jax.ShapeDtypeStruct((NP, NCLASS), _f32),
    )(xvp, dv9, h0, w, wout, bout.reshape(1, NCLASS))


def kernel(x, vertex, edges, degE, degV, W0, b0, W1, W2, Wout, bout):
    lamda, alpha = 0.5, 0.1
    vtx4 = vertex.astype(_i32).reshape(2, 16, NCH_T, CHUNK)
    edg4 = edges.astype(_i32).reshape(2, 16, NCH_T, CHUNK)

    dE = jnp.pad(degE.reshape(NE, 1), ((0, NEP - NE), (0, 0)))
    dv9 = jnp.pad((1.0 - alpha) * degV, ((0, NP - N), (0, 0)))
    xp = jnp.pad(x, ((0, NP - N), (0, 0)))

    h = _tc_input_layer(xp, W0, b0)
    h0 = h
    betas = [math.log(lamda / (i + 1) + 1) for i in range(2)]

    vtx3 = vtx4.reshape(2, 16, NCH_T * CHUNK)
    edg3 = edg4.reshape(2, 16, NCH_T * CHUNK)

    for i in range(2):
        xep = _sc_edge(h, vtx3, edg4)
        xe = _tc_edge_scale(xep, dE)
        xvp = _sc_node(xe, edg3, vtx4)
        if i == 0:
            h = _tc_gcnii_layer(xvp, dv9, h0, W1, betas[0])
        else:
            return _tc_gcnii_out_layer(xvp, dv9, h0, W2, betas[1],
                                       Wout, bout)[:N]


# speedup vs baseline: 6.8779x; 1.6638x over previous
import functools
import math

import jax
import jax.numpy as jnp
from jax import lax
from jax.experimental import pallas as pl
from jax.experimental.pallas import tpu as pltpu
from jax.experimental.pallas import tpu_sc as plsc

N = 10000
NP = 10240
NNZ = 320000
NE = 5000
NEP = 5120
NHID = 128
NCLASS = 40

CHUNK = 80
NCH_T = 125
ROW_BLK = 1024

_i32 = jnp.int32
_f32 = jnp.float32


def _zero_rows(zrow_v, dst_sh, row0, n16):
    def zb(k, _):
        pltpu.sync_copy(zrow_v, dst_sh.at[pl.ds(row0 + k * 16, 16)])
        return 0

    lax.fori_loop(0, n16, zb, 0)


def _init_zrow(zrow_v):
    def zrow_body(i, _):
        for j in range(8):
            zrow_v[i, pl.ds(j * 16, 16)] = jnp.zeros((16,), _f32)
        return 0

    lax.fori_loop(0, 16, zrow_body, 0)


def _pipelined_stage(data_hbm, acc_sh, gidx_v, sidx_v, rows, gsem, ssem):
    def start_g(i, b):
        pltpu.async_copy(data_hbm.at[gidx_v.at[pl.ds(i * CHUNK, CHUNK)]],
                         rows[b], gsem[b])

    def wait_g(b):
        pltpu.make_async_copy(data_hbm.at[gidx_v.at[pl.ds(0, CHUNK)]],
                              rows[b], gsem[b]).wait()

    def start_s(i, b):
        pltpu.async_copy(rows[b], acc_sh.at[sidx_v.at[i]], ssem[b], add=True)

    def wait_s(b):
        pltpu.make_async_copy(rows[b], acc_sh.at[sidx_v.at[0]], ssem[b]).wait()

    start_g(0, 0)

    def body(k, _):
        ia = 2 * k
        wait_g(0)

        @pl.when(k > 0)
        def _():
            wait_s(1)

        start_g(ia + 1, 1)
        start_s(ia, 0)
        wait_g(1)
        wait_s(0)
        start_g(ia + 2, 0)
        start_s(ia + 1, 1)
        return 0

    lax.fori_loop(0, (NCH_T - 1) // 2, body, 0)
    wait_g(0)
    wait_s(1)
    start_s(NCH_T - 1, 0)
    wait_s(0)


def _sc_stage_body(rows_total, data_hbm, gidx_hbm, sidx_hbm, out_hbm, acc_sh,
                   gidx_v, sidx_v, rows_a, rows_b, zrow_v, gsem_a, gsem_b,
                   ssem_a, ssem_b):
    c = lax.axis_index("c")
    s = lax.axis_index("s")
    rows_t = rows_total // 16
    _init_zrow(zrow_v)
    _zero_rows(zrow_v, acc_sh, s * rows_t, rows_t // 16)
    pltpu.sync_copy(gidx_hbm.at[c, s], gidx_v)
    pltpu.sync_copy(sidx_hbm.at[c, s], sidx_v)
    plsc.subcore_barrier()

    _pipelined_stage(data_hbm, acc_sh, gidx_v, sidx_v,
                     (rows_a, rows_b), (gsem_a, gsem_b), (ssem_a, ssem_b))

    plsc.subcore_barrier()
    pltpu.sync_copy(acc_sh.at[pl.ds(s * rows_t, rows_t)],
                    out_hbm.at[c, pl.ds(s * rows_t, rows_t)])


def _make_sc_stage(acc_rows):
    return functools.partial(
        pl.kernel,
        out_type=jax.ShapeDtypeStruct((2, acc_rows, NHID), _f32),
        mesh=plsc.VectorSubcoreMesh(core_axis_name="c", subcore_axis_name="s"),
        scratch_types=[
            pltpu.VMEM_SHARED((acc_rows, NHID), _f32),
            pltpu.VMEM((NCH_T * CHUNK,), _i32),
            pltpu.VMEM((NCH_T, CHUNK), _i32),
            pltpu.VMEM((CHUNK, NHID), _f32),
            pltpu.VMEM((CHUNK, NHID), _f32),
            pltpu.VMEM((16, NHID), _f32),
            pltpu.SemaphoreType.DMA,
            pltpu.SemaphoreType.DMA,
            pltpu.SemaphoreType.DMA,
            pltpu.SemaphoreType.DMA,
        ],
    )(functools.partial(_sc_stage_body, acc_rows))


_sc_edge = _make_sc_stage(NEP)
_sc_node = _make_sc_stage(NP)


def _k0_body(x_ref, w_ref, b_ref, o_ref):
    acc = jnp.dot(x_ref[...], w_ref[...], preferred_element_type=_f32)
    o_ref[...] = jnp.maximum(acc + b_ref[...], 0.0)


def _tc_input_layer(x, w0, b0):
    return pl.pallas_call(
        _k0_body,
        grid=(NP // ROW_BLK,),
        in_specs=[
            pl.BlockSpec((ROW_BLK, NHID), lambda i: (i, 0)),
            pl.BlockSpec((NHID, NHID), lambda i: (0, 0)),
            pl.BlockSpec((1, NHID), lambda i: (0, 0)),
        ],
        out_specs=pl.BlockSpec((ROW_BLK, NHID), lambda i: (i, 0)),
        out_shape=jax.ShapeDtypeStruct((NP, NHID), _f32),
    )(x, w0, b0.reshape(1, NHID))


def _kc_body(p_ref, d_ref, o_ref):
    d = d_ref[...]
    o_ref[...] = (p_ref[0] + p_ref[1]) * (d * d * d)


def _tc_edge_scale(xep, dE):
    return pl.pallas_call(
        _kc_body,
        grid=(NEP // ROW_BLK,),
        in_specs=[
            pl.BlockSpec((2, ROW_BLK, NHID), lambda i: (0, i, 0)),
            pl.BlockSpec((ROW_BLK, 1), lambda i: (i, 0)),
        ],
        out_specs=pl.BlockSpec((ROW_BLK, NHID), lambda i: (i, 0)),
        out_shape=jax.ShapeDtypeStruct((NEP, NHID), _f32),
    )(xep, dE)


def _k1_body(beta, xvp_ref, dv_ref, h0_ref, w_ref, o_ref):
    xv = xvp_ref[0] + xvp_ref[1]
    xi = xv * dv_ref[...] + 0.1 * h0_ref[...]
    acc = jnp.dot(xi, w_ref[...], preferred_element_type=_f32)
    o_ref[...] = jnp.maximum((1.0 - beta) * xi + beta * acc, 0.0)


def _tc_gcnii_layer(xvp, dv9, h0, w, beta):
    return pl.pallas_call(
        functools.partial(_k1_body, beta),
        grid=(NP // ROW_BLK,),
        in_specs=[
            pl.BlockSpec((2, ROW_BLK, NHID), lambda i: (0, i, 0)),
            pl.BlockSpec((ROW_BLK, 1), lambda i: (i, 0)),
            pl.BlockSpec((ROW_BLK, NHID), lambda i: (i, 0)),
            pl.BlockSpec((NHID, NHID), lambda i: (0, 0)),
        ],
        out_specs=pl.BlockSpec((ROW_BLK, NHID), lambda i: (i, 0)),
        out_shape=jax.ShapeDtypeStruct((NP, NHID), _f32),
    )(xvp, dv9, h0, w)


def _k1o_body(beta, xvp_ref, dv_ref, h0_ref, w_ref, wo_ref, bo_ref, o_ref):
    xv = xvp_ref[0] + xvp_ref[1]
    xi = xv * dv_ref[...] + 0.1 * h0_ref[...]
    acc = jnp.dot(xi, w_ref[...], preferred_element_type=_f32)
    h = jnp.maximum((1.0 - beta) * xi + beta * acc, 0.0)
    o_ref[...] = jnp.dot(h, wo_ref[...], preferred_element_type=_f32) + bo_ref[...]


def _tc_gcnii_out_layer(xvp, dv9, h0, w, beta, wout, bout):
    return pl.pallas_call(
        functools.partial(_k1o_body, beta),
        grid=(NP // ROW_BLK,),
        in_specs=[
            pl.BlockSpec((2, ROW_BLK, NHID), lambda i: (0, i, 0)),
            pl.BlockSpec((ROW_BLK, 1), lambda i: (i, 0)),
            pl.BlockSpec((ROW_BLK, NHID), lambda i: (i, 0)),
            pl.BlockSpec((NHID, NHID), lambda i: (0, 0)),
            pl.BlockSpec((NHID, NCLASS), lambda i: (0, 0)),
            pl.BlockSpec((1, NCLASS), lambda i: (0, 0)),
        ],
        out_specs=pl.BlockSpec((ROW_BLK, NCLASS), lambda i: (i, 0)),
        out_shape=jax.ShapeDtypeStruct((NP, NCLASS), _f32),
    )(xvp, dv9, h0, w, wout, bout.reshape(1, NCLASS))


def kernel(x, vertex, edges, degE, degV, W0, b0, W1, W2, Wout, bout):
    lamda, alpha = 0.5, 0.1
    vtx4 = vertex.astype(_i32).reshape(2, 16, NCH_T, CHUNK)
    edg4 = edges.astype(_i32).reshape(2, 16, NCH_T, CHUNK)

    dE = jnp.pad(degE.reshape(NE, 1), ((0, NEP - NE), (0, 0)))
    dv9 = jnp.pad((1.0 - alpha) * degV, ((0, NP - N), (0, 0)))
    xp = jnp.pad(x, ((0, NP - N), (0, 0)))

    h = _tc_input_layer(xp, W0, b0)
    h0 = h
    betas = [math.log(lamda / (i + 1) + 1) for i in range(2)]

    vtx3 = vtx4.reshape(2, 16, NCH_T * CHUNK)

    key = jnp.sort((vertex.astype(_i32) << 13) | edges.astype(_i32))
    sv4 = (key >> 13).reshape(2, 16, NCH_T, CHUNK)
    se = key & 8191
    se3 = se.reshape(2, 16, NCH_T * CHUNK)

    for i in range(2):
        xep = _sc_edge(h, vtx3, edg4)
        xe = _tc_edge_scale(xep, dE)
        xvp = _sc_node(xe, se3, sv4)
        if i == 0:
            h = _tc_gcnii_layer(xvp, dv9, h0, W1, betas[0])
        else:
            return _tc_gcnii_out_layer(xvp, dv9, h0, W2, betas[1],
                                       Wout, bout)[:N]


# speedup vs baseline: 8.9574x; 1.3023x over previous
import functools
import math

import jax
import jax.numpy as jnp
from jax import lax
from jax.experimental import pallas as pl
from jax.experimental.pallas import tpu as pltpu
from jax.experimental.pallas import tpu_sc as plsc

N = 10000
NP = 10240
NNZ = 320000
NE = 5000
NEP = 5120
NHID = 128
NCLASS = 40

CHUNK = 80
NCH_T = 125
ROW_BLK = 1024

_i32 = jnp.int32
_f32 = jnp.float32


def _zero_rows(zrow_v, dst_sh, row0, n16):
    def zb(k, _):
        pltpu.sync_copy(zrow_v, dst_sh.at[pl.ds(row0 + k * 16, 16)])
        return 0

    lax.fori_loop(0, n16, zb, 0)


def _init_zrow(zrow_v):
    def zrow_body(i, _):
        for j in range(8):
            zrow_v[i, pl.ds(j * 16, 16)] = jnp.zeros((16,), _f32)
        return 0

    lax.fori_loop(0, 16, zrow_body, 0)


def _pipelined_stage(data_hbm, acc_sh, gidx_v, sidx_v, rows, gsem, ssem):
    def start_g(i, b):
        pltpu.async_copy(data_hbm.at[gidx_v.at[pl.ds(i * CHUNK, CHUNK)]],
                         rows[b], gsem[b])

    def wait_g(b):
        pltpu.make_async_copy(data_hbm.at[gidx_v.at[pl.ds(0, CHUNK)]],
                              rows[b], gsem[b]).wait()

    def start_s(i, b):
        pltpu.async_copy(rows[b], acc_sh.at[sidx_v.at[i]], ssem[b], add=True)

    def wait_s(b):
        pltpu.make_async_copy(rows[b], acc_sh.at[sidx_v.at[0]], ssem[b]).wait()

    start_g(0, 0)

    def body(k, _):
        ia = 2 * k
        wait_g(0)

        @pl.when(k > 0)
        def _():
            wait_s(1)

        start_g(ia + 1, 1)
        start_s(ia, 0)
        wait_g(1)
        wait_s(0)
        start_g(ia + 2, 0)
        start_s(ia + 1, 1)
        return 0

    lax.fori_loop(0, (NCH_T - 1) // 2, body, 0)
    wait_g(0)
    wait_s(1)
    start_s(NCH_T - 1, 0)
    wait_s(0)


def _sc_stage_body(rows_total, data_hbm, gidx_hbm, sidx_hbm, out_hbm, acc_sh,
                   gidx_v, sidx_v, rows_a, rows_b, zrow_v, gsem_a, gsem_b,
                   ssem_a, ssem_b):
    c = lax.axis_index("c")
    s = lax.axis_index("s")
    rows_t = rows_total // 16
    _init_zrow(zrow_v)
    _zero_rows(zrow_v, acc_sh, s * rows_t, rows_t // 16)
    pltpu.sync_copy(gidx_hbm.at[c, s], gidx_v)
    pltpu.sync_copy(sidx_hbm.at[c, s], sidx_v)
    plsc.subcore_barrier()

    _pipelined_stage(data_hbm, acc_sh, gidx_v, sidx_v,
                     (rows_a, rows_b), (gsem_a, gsem_b), (ssem_a, ssem_b))

    plsc.subcore_barrier()
    pltpu.sync_copy(acc_sh.at[pl.ds(s * rows_t, rows_t)],
                    out_hbm.at[c, pl.ds(s * rows_t, rows_t)])


def _make_sc_stage(acc_rows):
    return functools.partial(
        pl.kernel,
        out_type=jax.ShapeDtypeStruct((2, acc_rows, NHID), _f32),
        mesh=plsc.VectorSubcoreMesh(core_axis_name="c", subcore_axis_name="s"),
        scratch_types=[
            pltpu.VMEM_SHARED((acc_rows, NHID), _f32),
            pltpu.VMEM((NCH_T * CHUNK,), _i32),
            pltpu.VMEM((NCH_T, CHUNK), _i32),
            pltpu.VMEM((CHUNK, NHID), _f32),
            pltpu.VMEM((CHUNK, NHID), _f32),
            pltpu.VMEM((16, NHID), _f32),
            pltpu.SemaphoreType.DMA,
            pltpu.SemaphoreType.DMA,
            pltpu.SemaphoreType.DMA,
            pltpu.SemaphoreType.DMA,
        ],
    )(functools.partial(_sc_stage_body, acc_rows))


_sc_edge = _make_sc_stage(NEP)
_sc_node = _make_sc_stage(NP)


def _k0_body(x_ref, w_ref, b_ref, o_ref):
    acc = jnp.dot(x_ref[...], w_ref[...], preferred_element_type=_f32)
    o_ref[...] = jnp.maximum(acc + b_ref[...], 0.0)


def _tc_input_layer(x, w0, b0):
    return pl.pallas_call(
        _k0_body,
        grid=(NP // ROW_BLK,),
        in_specs=[
            pl.BlockSpec((ROW_BLK, NHID), lambda i: (i, 0)),
            pl.BlockSpec((NHID, NHID), lambda i: (0, 0)),
            pl.BlockSpec((1, NHID), lambda i: (0, 0)),
        ],
        out_specs=pl.BlockSpec((ROW_BLK, NHID), lambda i: (i, 0)),
        out_shape=jax.ShapeDtypeStruct((NP, NHID), _f32),
    )(x, w0, b0.reshape(1, NHID))


def _kc_body(p_ref, d_ref, o_ref):
    d = d_ref[...]
    o_ref[...] = (p_ref[0] + p_ref[1]) * (d * d * d)


def _tc_edge_scale(xep, dE):
    return pl.pallas_call(
        _kc_body,
        grid=(NEP // ROW_BLK,),
        in_specs=[
            pl.BlockSpec((2, ROW_BLK, NHID), lambda i: (0, i, 0)),
            pl.BlockSpec((ROW_BLK, 1), lambda i: (i, 0)),
        ],
        out_specs=pl.BlockSpec((ROW_BLK, NHID), lambda i: (i, 0)),
        out_shape=jax.ShapeDtypeStruct((NEP, NHID), _f32),
    )(xep, dE)


def _k1_body(beta, xvp_ref, dv_ref, h0_ref, w_ref, o_ref):
    xv = xvp_ref[0] + xvp_ref[1]
    xi = xv * dv_ref[...] + 0.1 * h0_ref[...]
    acc = jnp.dot(xi, w_ref[...], preferred_element_type=_f32)
    o_ref[...] = jnp.maximum((1.0 - beta) * xi + beta * acc, 0.0)


def _tc_gcnii_layer(xvp, dv9, h0, w, beta):
    return pl.pallas_call(
        functools.partial(_k1_body, beta),
        grid=(NP // ROW_BLK,),
        in_specs=[
            pl.BlockSpec((2, ROW_BLK, NHID), lambda i: (0, i, 0)),
            pl.BlockSpec((ROW_BLK, 1), lambda i: (i, 0)),
            pl.BlockSpec((ROW_BLK, NHID), lambda i: (i, 0)),
            pl.BlockSpec((NHID, NHID), lambda i: (0, 0)),
        ],
        out_specs=pl.BlockSpec((ROW_BLK, NHID), lambda i: (i, 0)),
        out_shape=jax.ShapeDtypeStruct((NP, NHID), _f32),
    )(xvp, dv9, h0, w)


def _k1o_body(beta, xvp_ref, dv_ref, h0_ref, w_ref, wo_ref, bo_ref, o_ref):
    xv = xvp_ref[0] + xvp_ref[1]
    xi = xv * dv_ref[...] + 0.1 * h0_ref[...]
    acc = jnp.dot(xi, w_ref[...], preferred_element_type=_f32)
    h = jnp.maximum((1.0 - beta) * xi + beta * acc, 0.0)
    o_ref[...] = jnp.dot(h, wo_ref[...], preferred_element_type=_f32) + bo_ref[...]


def _tc_gcnii_out_layer(xvp, dv9, h0, w, beta, wout, bout):
    return pl.pallas_call(
        functools.partial(_k1o_body, beta),
        grid=(NP // ROW_BLK,),
        in_specs=[
            pl.BlockSpec((2, ROW_BLK, NHID), lambda i: (0, i, 0)),
            pl.BlockSpec((ROW_BLK, 1), lambda i: (i, 0)),
            pl.BlockSpec((ROW_BLK, NHID), lambda i: (i, 0)),
            pl.BlockSpec((NHID, NHID), lambda i: (0, 0)),
            pl.BlockSpec((NHID, NCLASS), lambda i: (0, 0)),
            pl.BlockSpec((1, NCLASS), lambda i: (0, 0)),
        ],
        out_specs=pl.BlockSpec((ROW_BLK, NCLASS), lambda i: (i, 0)),
        out_shape=jax.ShapeDtypeStruct((NP, NCLASS), _f32),
    )(xvp, dv9, h0, w, wout, bout.reshape(1, NCLASS))


def kernel(x, vertex, edges, degE, degV, W0, b0, W1, W2, Wout, bout):
    lamda, alpha = 0.5, 0.1
    vtx4 = vertex.astype(_i32).reshape(2, 16, NCH_T, CHUNK)
    edg4 = edges.astype(_i32).reshape(2, 16, NCH_T, CHUNK)

    dE = jnp.pad(degE.reshape(NE, 1), ((0, NEP - NE), (0, 0)))
    dv9 = jnp.pad((1.0 - alpha) * degV, ((0, NP - N), (0, 0)))
    xp = jnp.pad(x, ((0, NP - N), (0, 0)))

    h = _tc_input_layer(xp, W0, b0)
    h0 = h
    betas = [math.log(lamda / (i + 1) + 1) for i in range(2)]

    vtx3 = vtx4.reshape(2, 16, NCH_T * CHUNK)

    vt = vertex.astype(_i32).reshape(CHUNK, NNZ // CHUNK).T
    et = edges.astype(_i32).reshape(CHUNK, NNZ // CHUNK).T
    sv4 = vt.reshape(2, 16, NCH_T, CHUNK)
    se3 = et.reshape(2, 16, NCH_T * CHUNK)

    for i in range(2):
        xep = _sc_edge(h, vtx3, edg4)
        xe = _tc_edge_scale(xep, dE)
        xvp = _sc_node(xe, se3, sv4)
        if i == 0:
            h = _tc_gcnii_layer(xvp, dv9, h0, W1, betas[0])
        else:
            return _tc_gcnii_out_layer(xvp, dv9, h0, W2, betas[1],
                                       Wout, bout)[:N]


# speedup vs baseline: 9.1641x; 1.0231x over previous
import functools
import math

import jax
import jax.numpy as jnp
from jax import lax
from jax.experimental import pallas as pl
from jax.experimental.pallas import tpu as pltpu
from jax.experimental.pallas import tpu_sc as plsc

N = 10000
NP = 10240
NNZ = 320000
NE = 5000
NEP = 5120
NHID = 128
NCLASS = 40

CHUNK = 80
NCH_T = 125
ROW_BLK = 1024

_i32 = jnp.int32
_f32 = jnp.float32


def _zero_rows_async(zrow_v, dst_sh, row0, n16, sem):
    def zb(k, _):
        pltpu.async_copy(zrow_v, dst_sh.at[pl.ds(row0 + k * 16, 16)], sem)
        return 0

    lax.fori_loop(0, n16, zb, 0)

    def zw(k, _):
        pltpu.make_async_copy(zrow_v, dst_sh.at[pl.ds(row0, 16)], sem).wait()
        return 0

    lax.fori_loop(0, n16, zw, 0)


def _init_zrow(zrow_v):
    def zrow_body(i, _):
        for j in range(8):
            zrow_v[i, pl.ds(j * 16, 16)] = jnp.zeros((16,), _f32)
        return 0

    lax.fori_loop(0, 16, zrow_body, 0)


def _pipelined_stage(data_hbm, acc_sh, gidx_v, sidx_v, rows, gsem, ssem):
    def start_g(i, b):
        pltpu.async_copy(data_hbm.at[gidx_v.at[pl.ds(i * CHUNK, CHUNK)]],
                         rows[b], gsem[b])

    def wait_g(b):
        pltpu.make_async_copy(data_hbm.at[gidx_v.at[pl.ds(0, CHUNK)]],
                              rows[b], gsem[b]).wait()

    def start_s(i, b):
        pltpu.async_copy(rows[b], acc_sh.at[sidx_v.at[i]], ssem[b], add=True)

    def wait_s(b):
        pltpu.make_async_copy(rows[b], acc_sh.at[sidx_v.at[0]], ssem[b]).wait()

    start_g(0, 0)

    def body(k, _):
        ia = 2 * k
        wait_g(0)

        @pl.when(k > 0)
        def _():
            wait_s(1)

        start_g(ia + 1, 1)
        start_s(ia, 0)
        wait_g(1)
        wait_s(0)
        start_g(ia + 2, 0)
        start_s(ia + 1, 1)
        return 0

    lax.fori_loop(0, (NCH_T - 1) // 2, body, 0)
    wait_g(0)
    wait_s(1)
    start_s(NCH_T - 1, 0)
    wait_s(0)


def _sc_stage_body(rows_total, data_hbm, gidx_hbm, sidx_hbm, out_hbm, acc_sh,
                   gidx_v, sidx_v, rows_a, rows_b, zrow_v, gsem_a, gsem_b,
                   ssem_a, ssem_b):
    c = lax.axis_index("c")
    s = lax.axis_index("s")
    rows_t = rows_total // 16
    pltpu.async_copy(gidx_hbm.at[c, s], gidx_v, gsem_a)
    pltpu.async_copy(sidx_hbm.at[c, s], sidx_v, gsem_b)
    _init_zrow(zrow_v)
    _zero_rows_async(zrow_v, acc_sh, s * rows_t, rows_t // 16, ssem_a)
    pltpu.make_async_copy(gidx_hbm.at[c, s], gidx_v, gsem_a).wait()
    pltpu.make_async_copy(sidx_hbm.at[c, s], sidx_v, gsem_b).wait()
    plsc.subcore_barrier()

    _pipelined_stage(data_hbm, acc_sh, gidx_v, sidx_v,
                     (rows_a, rows_b), (gsem_a, gsem_b), (ssem_a, ssem_b))

    plsc.subcore_barrier()
    pltpu.sync_copy(acc_sh.at[pl.ds(s * rows_t, rows_t)],
                    out_hbm.at[c, pl.ds(s * rows_t, rows_t)])


def _make_sc_stage(acc_rows):
    return functools.partial(
        pl.kernel,
        out_type=jax.ShapeDtypeStruct((2, acc_rows, NHID), _f32),
        mesh=plsc.VectorSubcoreMesh(core_axis_name="c", subcore_axis_name="s"),
        scratch_types=[
            pltpu.VMEM_SHARED((acc_rows, NHID), _f32),
            pltpu.VMEM((NCH_T * CHUNK,), _i32),
            pltpu.VMEM((NCH_T, CHUNK), _i32),
            pltpu.VMEM((CHUNK, NHID), _f32),
            pltpu.VMEM((CHUNK, NHID), _f32),
            pltpu.VMEM((16, NHID), _f32),
            pltpu.SemaphoreType.DMA,
            pltpu.SemaphoreType.DMA,
            pltpu.SemaphoreType.DMA,
            pltpu.SemaphoreType.DMA,
        ],
    )(functools.partial(_sc_stage_body, acc_rows))


_sc_edge = _make_sc_stage(NEP)
_sc_node = _make_sc_stage(NP)


def _k0_body(x_ref, w_ref, b_ref, o_ref):
    acc = jnp.dot(x_ref[...], w_ref[...], preferred_element_type=_f32)
    o_ref[...] = jnp.maximum(acc + b_ref[...], 0.0)


def _tc_input_layer(x, w0, b0):
    return pl.pallas_call(
        _k0_body,
        grid=(NP // ROW_BLK,),
        in_specs=[
            pl.BlockSpec((ROW_BLK, NHID), lambda i: (i, 0)),
            pl.BlockSpec((NHID, NHID), lambda i: (0, 0)),
            pl.BlockSpec((1, NHID), lambda i: (0, 0)),
        ],
        out_specs=pl.BlockSpec((ROW_BLK, NHID), lambda i: (i, 0)),
        out_shape=jax.ShapeDtypeStruct((NP, NHID), _f32),
    )(x, w0, b0.reshape(1, NHID))


def _kc_body(p_ref, d_ref, o_ref):
    d = d_ref[...]
    o_ref[...] = (p_ref[0] + p_ref[1]) * (d * d * d)


def _tc_edge_scale(xep, dE):
    return pl.pallas_call(
        _kc_body,
        grid=(NEP // ROW_BLK,),
        in_specs=[
            pl.BlockSpec((2, ROW_BLK, NHID), lambda i: (0, i, 0)),
            pl.BlockSpec((ROW_BLK, 1), lambda i: (i, 0)),
        ],
        out_specs=pl.BlockSpec((ROW_BLK, NHID), lambda i: (i, 0)),
        out_shape=jax.ShapeDtypeStruct((NEP, NHID), _f32),
    )(xep, dE)


def _k1_body(beta, xvp_ref, dv_ref, h0_ref, w_ref, o_ref):
    xv = xvp_ref[0] + xvp_ref[1]
    xi = xv * dv_ref[...] + 0.1 * h0_ref[...]
    acc = jnp.dot(xi, w_ref[...], preferred_element_type=_f32)
    o_ref[...] = jnp.maximum((1.0 - beta) * xi + beta * acc, 0.0)


def _tc_gcnii_layer(xvp, dv9, h0, w, beta):
    return pl.pallas_call(
        functools.partial(_k1_body, beta),
        grid=(NP // ROW_BLK,),
        in_specs=[
            pl.BlockSpec((2, ROW_BLK, NHID), lambda i: (0, i, 0)),
            pl.BlockSpec((ROW_BLK, 1), lambda i: (i, 0)),
            pl.BlockSpec((ROW_BLK, NHID), lambda i: (i, 0)),
            pl.BlockSpec((NHID, NHID), lambda i: (0, 0)),
        ],
        out_specs=pl.BlockSpec((ROW_BLK, NHID), lambda i: (i, 0)),
        out_shape=jax.ShapeDtypeStruct((NP, NHID), _f32),
    )(xvp, dv9, h0, w)


def _k1o_body(beta, xvp_ref, dv_ref, h0_ref, w_ref, wo_ref, bo_ref, o_ref):
    xv = xvp_ref[0] + xvp_ref[1]
    xi = xv * dv_ref[...] + 0.1 * h0_ref[...]
    acc = jnp.dot(xi, w_ref[...], preferred_element_type=_f32)
    h = jnp.maximum((1.0 - beta) * xi + beta * acc, 0.0)
    o_ref[...] = jnp.dot(h, wo_ref[...], preferred_element_type=_f32) + bo_ref[...]


def _tc_gcnii_out_layer(xvp, dv9, h0, w, beta, wout, bout):
    return pl.pallas_call(
        functools.partial(_k1o_body, beta),
        grid=(NP // ROW_BLK,),
        in_specs=[
            pl.BlockSpec((2, ROW_BLK, NHID), lambda i: (0, i, 0)),
            pl.BlockSpec((ROW_BLK, 1), lambda i: (i, 0)),
            pl.BlockSpec((ROW_BLK, NHID), lambda i: (i, 0)),
            pl.BlockSpec((NHID, NHID), lambda i: (0, 0)),
            pl.BlockSpec((NHID, NCLASS), lambda i: (0, 0)),
            pl.BlockSpec((1, NCLASS), lambda i: (0, 0)),
        ],
        out_specs=pl.BlockSpec((ROW_BLK, NCLASS), lambda i: (i, 0)),
        out_shape=jax.ShapeDtypeStruct((NP, NCLASS), _f32),
    )(xvp, dv9, h0, w, wout, bout.reshape(1, NCLASS))


def kernel(x, vertex, edges, degE, degV, W0, b0, W1, W2, Wout, bout):
    lamda, alpha = 0.5, 0.1
    vtx4 = vertex.astype(_i32).reshape(2, 16, NCH_T, CHUNK)
    edg4 = edges.astype(_i32).reshape(2, 16, NCH_T, CHUNK)

    dE = jnp.pad(degE.reshape(NE, 1), ((0, NEP - NE), (0, 0)))
    dv9 = jnp.pad((1.0 - alpha) * degV, ((0, NP - N), (0, 0)))
    xp = jnp.pad(x, ((0, NP - N), (0, 0)))

    h = _tc_input_layer(xp, W0, b0)
    h0 = h
    betas = [math.log(lamda / (i + 1) + 1) for i in range(2)]

    vtx3 = vtx4.reshape(2, 16, NCH_T * CHUNK)

    vt = vertex.astype(_i32).reshape(CHUNK, NNZ // CHUNK).T
    et = edges.astype(_i32).reshape(CHUNK, NNZ // CHUNK).T
    sv4 = vt.reshape(2, 16, NCH_T, CHUNK)
    se3 = et.reshape(2, 16, NCH_T * CHUNK)

    for i in range(2):
        xep = _sc_edge(h, vtx3, edg4)
        xe = _tc_edge_scale(xep, dE)
        xvp = _sc_node(xe, se3, sv4)
        if i == 0:
            h = _tc_gcnii_layer(xvp, dv9, h0, W1, betas[0])
        else:
            return _tc_gcnii_out_layer(xvp, dv9, h0, W2, betas[1],
                                       Wout, bout)[:N]


# speedup vs baseline: 10.7761x; 1.1759x over previous
import functools
import math

import jax
import jax.numpy as jnp
from jax import lax
from jax.experimental import pallas as pl
from jax.experimental.pallas import tpu as pltpu
from jax.experimental.pallas import tpu_sc as plsc

N = 10000
NP = 10240
NNZ = 320000
NE = 5000
NEP = 5120
NHID = 128
NCLASS = 40

CHUNK = 80
NCH_T = 125
ROW_BLK = 1024

_i32 = jnp.int32
_f32 = jnp.float32


def _zero_rows_async(zrow_v, dst_sh, row0, n16, sem):
    def zb(k, _):
        pltpu.async_copy(zrow_v, dst_sh.at[pl.ds(row0 + k * 16, 16)], sem)
        return 0

    lax.fori_loop(0, n16, zb, 0)

    def zw(k, _):
        pltpu.make_async_copy(zrow_v, dst_sh.at[pl.ds(row0, 16)], sem).wait()
        return 0

    lax.fori_loop(0, n16, zw, 0)


def _init_zrow(zrow_v):
    def zrow_body(i, _):
        for j in range(8):
            zrow_v[i, pl.ds(j * 16, 16)] = jnp.zeros((16,), _f32)
        return 0

    lax.fori_loop(0, 16, zrow_body, 0)


def _pipelined_stage(data_hbm, acc_sh, gidx_v, sidx_v, rows, gsem, ssem):
    nb = len(rows)

    def start_g(i, b):
        pltpu.async_copy(data_hbm.at[gidx_v.at[pl.ds(i * CHUNK, CHUNK)]],
                         rows[b], gsem[b])

    def wait_g(b):
        pltpu.make_async_copy(data_hbm.at[gidx_v.at[pl.ds(0, CHUNK)]],
                              rows[b], gsem[b]).wait()

    def start_s(i, b):
        pltpu.async_copy(rows[b], acc_sh.at[sidx_v.at[i]], ssem[b], add=True)

    def wait_s(b):
        pltpu.make_async_copy(rows[b], acc_sh.at[sidx_v.at[0]], ssem[b]).wait()

    for b in range(nb - 1):
        start_g(b, b)

    def body(k, _):
        for b in range(nb):
            i = k * nb + b
            wait_g(b)
            start_s(i, b)

            @pl.when(i >= 1)
            def _():
                wait_s((b - 1) % nb)

            @pl.when(i + nb - 1 < NCH_T)
            def _():
                start_g(i + nb - 1, (b + nb - 1) % nb)
        return 0

    lax.fori_loop(0, NCH_T // nb, body, 0)
    for i in range(NCH_T - NCH_T % nb, NCH_T):
        b = i % nb
        wait_g(b)
        start_s(i, b)
        wait_s((b - 1) % nb)
    wait_s((NCH_T - 1) % nb)


def _sc_stage_body(rows_total, nbuf, data_hbm, gidx_hbm, sidx_hbm, out_hbm,
                   acc_sh, *scratch):
    rows = scratch[1:1 + nbuf]
    gsem = scratch[1 + nbuf + 1:1 + nbuf + 1 + nbuf]
    ssem = scratch[1 + nbuf + 1 + nbuf:]
    gidx_v = scratch[0][0]
    sidx_v = scratch[0][1]
    zrow_v = scratch[1 + nbuf]
    gsem_a, gsem_b = gsem[0], gsem[1]
    ssem_a = ssem[0]
    c = lax.axis_index("c")
    s = lax.axis_index("s")
    rows_t = rows_total // 16
    pltpu.async_copy(gidx_hbm.at[c, s], gidx_v, gsem_a)
    pltpu.async_copy(sidx_hbm.at[c, s], sidx_v, gsem_b)
    _init_zrow(zrow_v)
    _zero_rows_async(zrow_v, acc_sh, s * rows_t, rows_t // 16, ssem_a)
    pltpu.make_async_copy(gidx_hbm.at[c, s], gidx_v, gsem_a).wait()
    pltpu.make_async_copy(sidx_hbm.at[c, s], sidx_v, gsem_b).wait()
    plsc.subcore_barrier()

    _pipelined_stage(data_hbm, acc_sh, gidx_v, sidx_v, rows, gsem, ssem)

    plsc.subcore_barrier()
    pltpu.sync_copy(acc_sh.at[pl.ds(s * rows_t, rows_t)],
                    out_hbm.at[c, pl.ds(s * rows_t, rows_t)])


def _make_sc_stage(acc_rows, nbuf):
    return functools.partial(
        pl.kernel,
        out_type=jax.ShapeDtypeStruct((2, acc_rows, NHID), _f32),
        mesh=plsc.VectorSubcoreMesh(core_axis_name="c", subcore_axis_name="s"),
        scratch_types=[
            pltpu.VMEM_SHARED((acc_rows, NHID), _f32),
            [pltpu.VMEM((NCH_T * CHUNK,), _i32),
             pltpu.VMEM((NCH_T, CHUNK), _i32)],
        ] + [pltpu.VMEM((CHUNK, NHID), _f32) for _ in range(nbuf)] + [
            pltpu.VMEM((16, NHID), _f32),
        ] + [pltpu.SemaphoreType.DMA for _ in range(2 * nbuf)],
    )(functools.partial(_sc_stage_body, acc_rows, nbuf))


_sc_edge = _make_sc_stage(NEP, 4)
_sc_node = _make_sc_stage(NP, 2)


def _k0_body(x_ref, w_ref, b_ref, o_ref):
    acc = jnp.dot(x_ref[...], w_ref[...], preferred_element_type=_f32)
    o_ref[...] = jnp.maximum(acc + b_ref[...], 0.0)


def _tc_input_layer(x, w0, b0):
    return pl.pallas_call(
        _k0_body,
        grid=(NP // ROW_BLK,),
        in_specs=[
            pl.BlockSpec((ROW_BLK, NHID), lambda i: (i, 0)),
            pl.BlockSpec((NHID, NHID), lambda i: (0, 0)),
            pl.BlockSpec((1, NHID), lambda i: (0, 0)),
        ],
        out_specs=pl.BlockSpec((ROW_BLK, NHID), lambda i: (i, 0)),
        out_shape=jax.ShapeDtypeStruct((NP, NHID), _f32),
    )(x, w0, b0.reshape(1, NHID))


def _kc_body(p_ref, d_ref, o_ref):
    d = d_ref[...]
    o_ref[...] = (p_ref[0] + p_ref[1]) * (d * d * d)


def _tc_edge_scale(xep, dE):
    return pl.pallas_call(
        _kc_body,
        grid=(NEP // ROW_BLK,),
        in_specs=[
            pl.BlockSpec((2, ROW_BLK, NHID), lambda i: (0, i, 0)),
            pl.BlockSpec((ROW_BLK, 1), lambda i: (i, 0)),
        ],
        out_specs=pl.BlockSpec((ROW_BLK, NHID), lambda i: (i, 0)),
        out_shape=jax.ShapeDtypeStruct((NEP, NHID), _f32),
    )(xep, dE)


def _k1_body(beta, xvp_ref, dv_ref, h0_ref, w_ref, o_ref):
    xv = xvp_ref[0] + xvp_ref[1]
    xi = xv * dv_ref[...] + 0.1 * h0_ref[...]
    acc = jnp.dot(xi, w_ref[...], preferred_element_type=_f32)
    o_ref[...] = jnp.maximum((1.0 - beta) * xi + beta * acc, 0.0)


def _tc_gcnii_layer(xvp, dv9, h0, w, beta):
    return pl.pallas_call(
        functools.partial(_k1_body, beta),
        grid=(NP // ROW_BLK,),
        in_specs=[
            pl.BlockSpec((2, ROW_BLK, NHID), lambda i: (0, i, 0)),
            pl.BlockSpec((ROW_BLK, 1), lambda i: (i, 0)),
            pl.BlockSpec((ROW_BLK, NHID), lambda i: (i, 0)),
            pl.BlockSpec((NHID, NHID), lambda i: (0, 0)),
        ],
        out_specs=pl.BlockSpec((ROW_BLK, NHID), lambda i: (i, 0)),
        out_shape=jax.ShapeDtypeStruct((NP, NHID), _f32),
    )(xvp, dv9, h0, w)


def _k1o_body(beta, xvp_ref, dv_ref, h0_ref, w_ref, wo_ref, bo_ref, o_ref):
    xv = xvp_ref[0] + xvp_ref[1]
    xi = xv * dv_ref[...] + 0.1 * h0_ref[...]
    acc = jnp.dot(xi, w_ref[...], preferred_element_type=_f32)
    h = jnp.maximum((1.0 - beta) * xi + beta * acc, 0.0)
    o_ref[...] = jnp.dot(h, wo_ref[...], preferred_element_type=_f32) + bo_ref[...]


def _tc_gcnii_out_layer(xvp, dv9, h0, w, beta, wout, bout):
    return pl.pallas_call(
        functools.partial(_k1o_body, beta),
        grid=(NP // ROW_BLK,),
        in_specs=[
            pl.BlockSpec((2, ROW_BLK, NHID), lambda i: (0, i, 0)),
            pl.BlockSpec((ROW_BLK, 1), lambda i: (i, 0)),
            pl.BlockSpec((ROW_BLK, NHID), lambda i: (i, 0)),
            pl.BlockSpec((NHID, NHID), lambda i: (0, 0)),
            pl.BlockSpec((NHID, NCLASS), lambda i: (0, 0)),
            pl.BlockSpec((1, NCLASS), lambda i: (0, 0)),
        ],
        out_specs=pl.BlockSpec((ROW_BLK, NCLASS), lambda i: (i, 0)),
        out_shape=jax.ShapeDtypeStruct((NP, NCLASS), _f32),
    )(xvp, dv9, h0, w, wout, bout.reshape(1, NCLASS))


def kernel(x, vertex, edges, degE, degV, W0, b0, W1, W2, Wout, bout):
    lamda, alpha = 0.5, 0.1
    vtx4 = vertex.astype(_i32).reshape(2, 16, NCH_T, CHUNK)
    edg4 = edges.astype(_i32).reshape(2, 16, NCH_T, CHUNK)

    dE = jnp.pad(degE.reshape(NE, 1), ((0, NEP - NE), (0, 0)))
    dv9 = jnp.pad((1.0 - alpha) * degV, ((0, NP - N), (0, 0)))
    xp = jnp.pad(x, ((0, NP - N), (0, 0)))

    h = _tc_input_layer(xp, W0, b0)
    h0 = h
    betas = [math.log(lamda / (i + 1) + 1) for i in range(2)]

    vtx3 = vtx4.reshape(2, 16, NCH_T * CHUNK)

    vt = vertex.astype(_i32).reshape(CHUNK, NNZ // CHUNK).T
    et = edges.astype(_i32).reshape(CHUNK, NNZ // CHUNK).T
    sv4 = vt.reshape(2, 16, NCH_T, CHUNK)
    se3 = et.reshape(2, 16, NCH_T * CHUNK)

    for i in range(2):
        xep = _sc_edge(h, vtx3, edg4)
        xe = _tc_edge_scale(xep, dE)
        xvp = _sc_node(xe, se3, sv4)
        if i == 0:
            h = _tc_gcnii_layer(xvp, dv9, h0, W1, betas[0])
        else:
            return _tc_gcnii_out_layer(xvp, dv9, h0, W2, betas[1],
                                       Wout, bout)[:N]


# speedup vs baseline: 13.6760x; 1.2691x over previous
import functools
import math

import jax
import jax.numpy as jnp
from jax import lax
from jax.experimental import pallas as pl
from jax.experimental.pallas import tpu as pltpu
from jax.experimental.pallas import tpu_sc as plsc

N = 10000
NP = 10240
NNZ = 320000
NE = 5000
NEP = 5120
NHID = 128
NCLASS = 40

CHUNK = 80
NCH_T = 125
ROW_BLK = 1024

_i32 = jnp.int32
_f32 = jnp.float32


def _zero_rows_async(zrow_v, dst_sh, row0, n16, sem):
    def zb(k, _):
        pltpu.async_copy(zrow_v, dst_sh.at[pl.ds(row0 + k * 8, 8)], sem)
        return 0

    lax.fori_loop(0, n16, zb, 0)

    def zw(k, _):
        pltpu.make_async_copy(zrow_v, dst_sh.at[pl.ds(row0, 8)], sem).wait()
        return 0

    lax.fori_loop(0, n16, zw, 0)


def _init_zrow(zrow_v):
    def zrow_body(i, _):
        for j in range(8):
            zrow_v[i, pl.ds(j * 16, 16)] = jnp.zeros((16,), _f32)
        return 0

    lax.fori_loop(0, 8, zrow_body, 0)


def _sc_stage_body(rows_total, nbuf, data_hbm, gidx_hbm, sidx_hbm, out_hbm,
                   acc_sh, *scratch):
    sidx_v = scratch[0]
    ibufs = scratch[1:1 + nbuf]
    rows = scratch[1 + nbuf:1 + 2 * nbuf]
    zrow_v = scratch[1 + 2 * nbuf]
    sems = scratch[2 + 2 * nbuf:]
    gsem = sems[0:nbuf]
    ssem = sems[nbuf:2 * nbuf]
    isem = sems[2 * nbuf:3 * nbuf]
    c = lax.axis_index("c")
    s = lax.axis_index("s")
    rows_t = rows_total // 16

    gbase = (c * 16 + s) * (NCH_T * CHUNK)

    def start_i(i, b):
        pltpu.async_copy(gidx_hbm.at[pl.ds(gbase + i * CHUNK, CHUNK)],
                         ibufs[b], isem[b])

    def wait_i(b):
        pltpu.make_async_copy(gidx_hbm.at[pl.ds(0, CHUNK)],
                              ibufs[b], isem[b]).wait()

    def start_g(i, b):
        pltpu.async_copy(data_hbm.at[ibufs[b]], rows[b], gsem[b])

    def wait_g(b):
        pltpu.make_async_copy(data_hbm.at[ibufs[b]], rows[b], gsem[b]).wait()

    def start_s(i, b):
        pltpu.async_copy(rows[b], acc_sh.at[sidx_v.at[i]], ssem[b], add=True)

    def wait_s(b):
        pltpu.make_async_copy(rows[b], acc_sh.at[sidx_v.at[0]], ssem[b]).wait()

    pltpu.async_copy(sidx_hbm.at[c, s], sidx_v, gsem[0])
    for b in range(nbuf):
        start_i(b, b)
    _init_zrow(zrow_v)
    _zero_rows_async(zrow_v, acc_sh, s * rows_t, rows_t // 8, ssem[0])
    pltpu.make_async_copy(sidx_hbm.at[c, s], sidx_v, gsem[0]).wait()
    plsc.subcore_barrier()

    for b in range(nbuf - 1):
        wait_i(b)
        start_g(b, b)

    def body(k, _):
        for b in range(nbuf):
            i = k * nbuf + b
            wait_g(b)

            @pl.when(i + nbuf < NCH_T)
            def _():
                start_i(i + nbuf, b)

            start_s(i, b)

            @pl.when(i >= 1)
            def _():
                wait_s((b - 1) % nbuf)

            @pl.when(i + nbuf - 1 < NCH_T)
            def _():
                wait_i((b - 1) % nbuf)
                start_g(i + nbuf - 1, (b - 1) % nbuf)
        return 0

    lax.fori_loop(0, NCH_T // nbuf, body, 0)
    for i in range(NCH_T - NCH_T % nbuf, NCH_T):
        b = i % nbuf
        wait_g(b)
        start_s(i, b)
        wait_s((b - 1) % nbuf)
    wait_s((NCH_T - 1) % nbuf)

    plsc.subcore_barrier()
    pltpu.sync_copy(acc_sh.at[pl.ds(s * rows_t, rows_t)],
                    out_hbm.at[c, pl.ds(s * rows_t, rows_t)])


def _make_sc_stage(acc_rows, nbuf):
    return functools.partial(
        pl.kernel,
        out_type=jax.ShapeDtypeStruct((2, acc_rows, NHID), _f32),
        mesh=plsc.VectorSubcoreMesh(core_axis_name="c", subcore_axis_name="s"),
        scratch_types=[
            pltpu.VMEM_SHARED((acc_rows, NHID), _f32),
            pltpu.VMEM((NCH_T, CHUNK), _i32),
        ] + [pltpu.VMEM((CHUNK,), _i32) for _ in range(nbuf)]
          + [pltpu.VMEM((CHUNK, NHID), _f32) for _ in range(nbuf)] + [
            pltpu.VMEM((8, NHID), _f32),
        ] + [pltpu.SemaphoreType.DMA for _ in range(3 * nbuf)],
    )(functools.partial(_sc_stage_body, acc_rows, nbuf))


_sc_edge = _make_sc_stage(NEP, 6)
_sc_node = _make_sc_stage(NP, 3)


def _k0_body(x_ref, w_ref, b_ref, o_ref):
    acc = jnp.dot(x_ref[...], w_ref[...], preferred_element_type=_f32)
    o_ref[...] = jnp.maximum(acc + b_ref[...], 0.0)


def _tc_input_layer(x, w0, b0):
    return pl.pallas_call(
        _k0_body,
        grid=(NP // ROW_BLK,),
        in_specs=[
            pl.BlockSpec((ROW_BLK, NHID), lambda i: (i, 0)),
            pl.BlockSpec((NHID, NHID), lambda i: (0, 0)),
            pl.BlockSpec((1, NHID), lambda i: (0, 0)),
        ],
        out_specs=pl.BlockSpec((ROW_BLK, NHID), lambda i: (i, 0)),
        out_shape=jax.ShapeDtypeStruct((NP, NHID), _f32),
    )(x, w0, b0.reshape(1, NHID))


def _kc_body(p_ref, d_ref, o_ref):
    d = d_ref[...]
    o_ref[...] = (p_ref[0] + p_ref[1]) * (d * d * d)


def _tc_edge_scale(xep, dE):
    return pl.pallas_call(
        _kc_body,
        grid=(NEP // ROW_BLK,),
        in_specs=[
            pl.BlockSpec((2, ROW_BLK, NHID), lambda i: (0, i, 0)),
            pl.BlockSpec((ROW_BLK, 1), lambda i: (i, 0)),
        ],
        out_specs=pl.BlockSpec((ROW_BLK, NHID), lambda i: (i, 0)),
        out_shape=jax.ShapeDtypeStruct((NEP, NHID), _f32),
    )(xep, dE)


def _k1_body(beta, xvp_ref, dv_ref, h0_ref, w_ref, o_ref):
    xv = xvp_ref[0] + xvp_ref[1]
    xi = xv * dv_ref[...] + 0.1 * h0_ref[...]
    acc = jnp.dot(xi, w_ref[...], preferred_element_type=_f32)
    o_ref[...] = jnp.maximum((1.0 - beta) * xi + beta * acc, 0.0)


def _tc_gcnii_layer(xvp, dv9, h0, w, beta):
    return pl.pallas_call(
        functools.partial(_k1_body, beta),
        grid=(NP // ROW_BLK,),
        in_specs=[
            pl.BlockSpec((2, ROW_BLK, NHID), lambda i: (0, i, 0)),
            pl.BlockSpec((ROW_BLK, 1), lambda i: (i, 0)),
            pl.BlockSpec((ROW_BLK, NHID), lambda i: (i, 0)),
            pl.BlockSpec((NHID, NHID), lambda i: (0, 0)),
        ],
        out_specs=pl.BlockSpec((ROW_BLK, NHID), lambda i: (i, 0)),
        out_shape=jax.ShapeDtypeStruct((NP, NHID), _f32),
    )(xvp, dv9, h0, w)


def _k1o_body(beta, xvp_ref, dv_ref, h0_ref, w_ref, wo_ref, bo_ref, o_ref):
    xv = xvp_ref[0] + xvp_ref[1]
    xi = xv * dv_ref[...] + 0.1 * h0_ref[...]
    acc = jnp.dot(xi, w_ref[...], preferred_element_type=_f32)
    h = jnp.maximum((1.0 - beta) * xi + beta * acc, 0.0)
    o_ref[...] = jnp.dot(h, wo_ref[...], preferred_element_type=_f32) + bo_ref[...]


def _tc_gcnii_out_layer(xvp, dv9, h0, w, beta, wout, bout):
    return pl.pallas_call(
        functools.partial(_k1o_body, beta),
        grid=(NP // ROW_BLK,),
        in_specs=[
            pl.BlockSpec((2, ROW_BLK, NHID), lambda i: (0, i, 0)),
            pl.BlockSpec((ROW_BLK, 1), lambda i: (i, 0)),
            pl.BlockSpec((ROW_BLK, NHID), lambda i: (i, 0)),
            pl.BlockSpec((NHID, NHID), lambda i: (0, 0)),
            pl.BlockSpec((NHID, NCLASS), lambda i: (0, 0)),
            pl.BlockSpec((1, NCLASS), lambda i: (0, 0)),
        ],
        out_specs=pl.BlockSpec((ROW_BLK, NCLASS), lambda i: (i, 0)),
        out_shape=jax.ShapeDtypeStruct((NP, NCLASS), _f32),
    )(xvp, dv9, h0, w, wout, bout.reshape(1, NCLASS))


def kernel(x, vertex, edges, degE, degV, W0, b0, W1, W2, Wout, bout):
    lamda, alpha = 0.5, 0.1
    vtx4 = vertex.astype(_i32).reshape(2, 16, NCH_T, CHUNK)
    edg4 = edges.astype(_i32).reshape(2, 16, NCH_T, CHUNK)

    dE = jnp.pad(degE.reshape(NE, 1), ((0, NEP - NE), (0, 0)))
    dv9 = jnp.pad((1.0 - alpha) * degV, ((0, NP - N), (0, 0)))
    xp = jnp.pad(x, ((0, NP - N), (0, 0)))

    h = _tc_input_layer(xp, W0, b0)
    h0 = h
    betas = [math.log(lamda / (i + 1) + 1) for i in range(2)]

    vtx1 = vtx4.reshape(NNZ)

    vt = vertex.astype(_i32).reshape(CHUNK, NNZ // CHUNK).T
    et = edges.astype(_i32).reshape(CHUNK, NNZ // CHUNK).T
    sv4 = vt.reshape(2, 16, NCH_T, CHUNK)
    se1 = et.reshape(NNZ)

    for i in range(2):
        xep = _sc_edge(h, vtx1, edg4)
        xe = _tc_edge_scale(xep, dE)
        xvp = _sc_node(xe, se1, sv4)
        if i == 0:
            h = _tc_gcnii_layer(xvp, dv9, h0, W1, betas[0])
        else:
            return _tc_gcnii_out_layer(xvp, dv9, h0, W2, betas[1],
                                       Wout, bout)[:N]


# speedup vs baseline: 13.8230x; 1.0107x over previous
import functools
import math

import jax
import jax.numpy as jnp
from jax import lax
from jax.experimental import pallas as pl
from jax.experimental.pallas import tpu as pltpu
from jax.experimental.pallas import tpu_sc as plsc

N = 10000
NP = 10240
NNZ = 320000
NE = 5000
NEP = 5120
NHID = 128
NCLASS = 40

CHUNK = 80
NCH_T = 125
ROW_BLK = 1024

_i32 = jnp.int32
_f32 = jnp.float32


def _zero_rows_async(zrow_v, dst_sh, row0, n16, sem):
    def zb(k, _):
        pltpu.async_copy(zrow_v, dst_sh.at[pl.ds(row0 + k * 8, 8)], sem)
        return 0

    lax.fori_loop(0, n16, zb, 0)

    def zw(k, _):
        pltpu.make_async_copy(zrow_v, dst_sh.at[pl.ds(row0, 8)], sem).wait()
        return 0

    lax.fori_loop(0, n16, zw, 0)


def _init_zrow(zrow_v):
    def zrow_body(i, _):
        for j in range(8):
            zrow_v[i, pl.ds(j * 16, 16)] = jnp.zeros((16,), _f32)
        return 0

    lax.fori_loop(0, 8, zrow_body, 0)


def _sc_stage_body(rows_total, nbuf, data_hbm, gidx_hbm, sidx_hbm, out_hbm,
                   acc_sh, *scratch):
    ibufs = scratch[0:nbuf]
    sbufs = scratch[nbuf:2 * nbuf]
    rows = scratch[2 * nbuf:3 * nbuf]
    zrow_v = scratch[3 * nbuf]
    sems = scratch[1 + 3 * nbuf:]
    gsem = sems[0:nbuf]
    ssem = sems[nbuf:2 * nbuf]
    isem = sems[2 * nbuf:3 * nbuf]
    sisem = sems[3 * nbuf:4 * nbuf]
    c = lax.axis_index("c")
    s = lax.axis_index("s")
    rows_t = rows_total // 16

    gbase = (c * 16 + s) * (NCH_T * CHUNK)

    def start_i(i, b):
        pltpu.async_copy(gidx_hbm.at[pl.ds(gbase + i * CHUNK, CHUNK)],
                         ibufs[b], isem[b])

    def wait_i(b):
        pltpu.make_async_copy(gidx_hbm.at[pl.ds(0, CHUNK)],
                              ibufs[b], isem[b]).wait()

    def start_g(i, b):
        pltpu.async_copy(data_hbm.at[ibufs[b]], rows[b], gsem[b])

    def wait_g(b):
        pltpu.make_async_copy(data_hbm.at[ibufs[b]], rows[b], gsem[b]).wait()

    def start_si(i, b):
        pltpu.async_copy(sidx_hbm.at[pl.ds(gbase + i * CHUNK, CHUNK)],
                         sbufs[b], sisem[b])

    def wait_si(b):
        pltpu.make_async_copy(sidx_hbm.at[pl.ds(0, CHUNK)],
                              sbufs[b], sisem[b]).wait()

    def start_s(i, b):
        wait_si(b)
        pltpu.async_copy(rows[b], acc_sh.at[sbufs[b]], ssem[b], add=True)

    def wait_s(b):
        pltpu.make_async_copy(rows[b], acc_sh.at[sbufs[0]], ssem[b]).wait()

    for b in range(nbuf):
        start_i(b, b)
        start_si(b, b)
    _init_zrow(zrow_v)
    _zero_rows_async(zrow_v, acc_sh, s * rows_t, rows_t // 8, ssem[0])
    plsc.subcore_barrier()

    for b in range(nbuf - 1):
        wait_i(b)
        start_g(b, b)

    def body(k, _):
        for b in range(nbuf):
            i = k * nbuf + b
            wait_g(b)

            @pl.when(i + nbuf < NCH_T)
            def _():
                start_i(i + nbuf, b)

            start_s(i, b)

            @pl.when(i >= 1)
            def _():
                wait_s((b - 1) % nbuf)

                @pl.when(i - 1 + nbuf < NCH_T)
                def _():
                    start_si(i - 1 + nbuf, (b - 1) % nbuf)

            @pl.when(i + nbuf - 1 < NCH_T)
            def _():
                wait_i((b - 1) % nbuf)
                start_g(i + nbuf - 1, (b - 1) % nbuf)
        return 0

    lax.fori_loop(0, NCH_T // nbuf, body, 0)
    for i in range(NCH_T - NCH_T % nbuf, NCH_T):
        b = i % nbuf
        wait_g(b)
        start_s(i, b)
        wait_s((b - 1) % nbuf)
    wait_s((NCH_T - 1) % nbuf)

    plsc.subcore_barrier()
    pltpu.sync_copy(acc_sh.at[pl.ds(s * rows_t, rows_t)],
                    out_hbm.at[c, pl.ds(s * rows_t, rows_t)])


def _make_sc_stage(acc_rows, nbuf):
    return functools.partial(
        pl.kernel,
        out_type=jax.ShapeDtypeStruct((2, acc_rows, NHID), _f32),
        mesh=plsc.VectorSubcoreMesh(core_axis_name="c", subcore_axis_name="s"),
        scratch_types=[
            pltpu.VMEM_SHARED((acc_rows, NHID), _f32),
        ] + [pltpu.VMEM((CHUNK,), _i32) for _ in range(2 * nbuf)]
          + [pltpu.VMEM((CHUNK, NHID), _f32) for _ in range(nbuf)] + [
            pltpu.VMEM((8, NHID), _f32),
        ] + [pltpu.SemaphoreType.DMA for _ in range(4 * nbuf)],
    )(functools.partial(_sc_stage_body, acc_rows, nbuf))


_sc_edge = _make_sc_stage(NEP, 6)
_sc_node = _make_sc_stage(NP, 3)


def _k0_body(x_ref, w_ref, b_ref, o_ref):
    acc = jnp.dot(x_ref[...], w_ref[...], preferred_element_type=_f32)
    o_ref[...] = jnp.maximum(acc + b_ref[...], 0.0)


def _tc_input_layer(x, w0, b0):
    return pl.pallas_call(
        _k0_body,
        grid=(NP // ROW_BLK,),
        in_specs=[
            pl.BlockSpec((ROW_BLK, NHID), lambda i: (i, 0)),
            pl.BlockSpec((NHID, NHID), lambda i: (0, 0)),
            pl.BlockSpec((1, NHID), lambda i: (0, 0)),
        ],
        out_specs=pl.BlockSpec((ROW_BLK, NHID), lambda i: (i, 0)),
        out_shape=jax.ShapeDtypeStruct((NP, NHID), _f32),
    )(x, w0, b0.reshape(1, NHID))


def _kc_body(p_ref, d_ref, o_ref):
    d = d_ref[...]
    o_ref[...] = (p_ref[0] + p_ref[1]) * (d * d * d)


def _tc_edge_scale(xep, dE):
    return pl.pallas_call(
        _kc_body,
        grid=(NEP // ROW_BLK,),
        in_specs=[
            pl.BlockSpec((2, ROW_BLK, NHID), lambda i: (0, i, 0)),
            pl.BlockSpec((ROW_BLK, 1), lambda i: (i, 0)),
        ],
        out_specs=pl.BlockSpec((ROW_BLK, NHID), lambda i: (i, 0)),
        out_shape=jax.ShapeDtypeStruct((NEP, NHID), _f32),
    )(xep, dE)


def _k1_body(beta, xvp_ref, dv_ref, h0_ref, w_ref, o_ref):
    xv = xvp_ref[0] + xvp_ref[1]
    xi = xv * dv_ref[...] + 0.1 * h0_ref[...]
    acc = jnp.dot(xi, w_ref[...], preferred_element_type=_f32)
    o_ref[...] = jnp.maximum((1.0 - beta) * xi + beta * acc, 0.0)


def _tc_gcnii_layer(xvp, dv9, h0, w, beta):
    return pl.pallas_call(
        functools.partial(_k1_body, beta),
        grid=(NP // ROW_BLK,),
        in_specs=[
            pl.BlockSpec((2, ROW_BLK, NHID), lambda i: (0, i, 0)),
            pl.BlockSpec((ROW_BLK, 1), lambda i: (i, 0)),
            pl.BlockSpec((ROW_BLK, NHID), lambda i: (i, 0)),
            pl.BlockSpec((NHID, NHID), lambda i: (0, 0)),
        ],
        out_specs=pl.BlockSpec((ROW_BLK, NHID), lambda i: (i, 0)),
        out_shape=jax.ShapeDtypeStruct((NP, NHID), _f32),
    )(xvp, dv9, h0, w)


def _k1o_body(beta, xvp_ref, dv_ref, h0_ref, w_ref, wo_ref, bo_ref, o_ref):
    xv = xvp_ref[0] + xvp_ref[1]
    xi = xv * dv_ref[...] + 0.1 * h0_ref[...]
    acc = jnp.dot(xi, w_ref[...], preferred_element_type=_f32)
    h = jnp.maximum((1.0 - beta) * xi + beta * acc, 0.0)
    o_ref[...] = jnp.dot(h, wo_ref[...], preferred_element_type=_f32) + bo_ref[...]


def _tc_gcnii_out_layer(xvp, dv9, h0, w, beta, wout, bout):
    return pl.pallas_call(
        functools.partial(_k1o_body, beta),
        grid=(NP // ROW_BLK,),
        in_specs=[
            pl.BlockSpec((2, ROW_BLK, NHID), lambda i: (0, i, 0)),
            pl.BlockSpec((ROW_BLK, 1), lambda i: (i, 0)),
            pl.BlockSpec((ROW_BLK, NHID), lambda i: (i, 0)),
            pl.BlockSpec((NHID, NHID), lambda i: (0, 0)),
            pl.BlockSpec((NHID, NCLASS), lambda i: (0, 0)),
            pl.BlockSpec((1, NCLASS), lambda i: (0, 0)),
        ],
        out_specs=pl.BlockSpec((ROW_BLK, NCLASS), lambda i: (i, 0)),
        out_shape=jax.ShapeDtypeStruct((NP, NCLASS), _f32),
    )(xvp, dv9, h0, w, wout, bout.reshape(1, NCLASS))


def kernel(x, vertex, edges, degE, degV, W0, b0, W1, W2, Wout, bout):
    lamda, alpha = 0.5, 0.1
    edg1 = edges.astype(_i32)

    dE = jnp.pad(degE.reshape(NE, 1), ((0, NEP - NE), (0, 0)))
    dv9 = jnp.pad((1.0 - alpha) * degV, ((0, NP - N), (0, 0)))
    xp = jnp.pad(x, ((0, NP - N), (0, 0)))

    h = _tc_input_layer(xp, W0, b0)
    h0 = h
    betas = [math.log(lamda / (i + 1) + 1) for i in range(2)]

    vtx1 = vertex.astype(_i32)

    vt = vertex.astype(_i32).reshape(CHUNK, NNZ // CHUNK).T
    et = edges.astype(_i32).reshape(CHUNK, NNZ // CHUNK).T
    sv1 = vt.reshape(NNZ)
    se1 = et.reshape(NNZ)

    for i in range(2):
        xep = _sc_edge(h, vtx1, edg1)
        xe = _tc_edge_scale(xep, dE)
        xvp = _sc_node(xe, se1, sv1)
        if i == 0:
            h = _tc_gcnii_layer(xvp, dv9, h0, W1, betas[0])
        else:
            return _tc_gcnii_out_layer(xvp, dv9, h0, W2, betas[1],
                                       Wout, bout)[:N]


# speedup vs baseline: 13.8733x; 1.0036x over previous
import functools
import math

import jax
import jax.numpy as jnp
from jax import lax
from jax.experimental import pallas as pl
from jax.experimental.pallas import tpu as pltpu
from jax.experimental.pallas import tpu_sc as plsc

N = 10000
NP = 10240
NNZ = 320000
NE = 5000
NEP = 5120
NHID = 128
NCLASS = 40

CHUNK = 80
NCH_T = 125
ROW_BLK = 1024

_i32 = jnp.int32
_f32 = jnp.float32


def _zero_rows_async(zrow_v, dst_sh, row0, n16, sem):
    def zb(k, _):
        pltpu.async_copy(zrow_v, dst_sh.at[pl.ds(row0 + k * 8, 8)], sem)
        return 0

    lax.fori_loop(0, n16, zb, 0)

    def zw(k, _):
        pltpu.make_async_copy(zrow_v, dst_sh.at[pl.ds(row0, 8)], sem).wait()
        return 0

    lax.fori_loop(0, n16, zw, 0)


def _init_zrow(zrow_v):
    def zrow_body(i, _):
        for j in range(8):
            zrow_v[i, pl.ds(j * 16, 16)] = jnp.zeros((16,), _f32)
        return 0

    lax.fori_loop(0, 8, zrow_body, 0)


def _sc_stage_body(rows_total, nbuf, data_hbm, gidx_hbm, sidx_hbm, out_hbm,
                   acc_sh, *scratch):
    ibufs = scratch[0:nbuf]
    sbufs = scratch[nbuf:2 * nbuf]
    rows = scratch[2 * nbuf:3 * nbuf]
    zrow_v = scratch[3 * nbuf]
    sems = scratch[1 + 3 * nbuf:]
    gsem = sems[0:nbuf]
    ssem = sems[nbuf:2 * nbuf]
    isem = sems[2 * nbuf:3 * nbuf]
    sisem = sems[3 * nbuf:4 * nbuf]
    c = lax.axis_index("c")
    s = lax.axis_index("s")
    rows_t = rows_total // 16

    gbase = (c * 16 + s) * (NCH_T * CHUNK)

    def start_i(i, b):
        pltpu.async_copy(gidx_hbm.at[pl.ds(gbase + i * CHUNK, CHUNK)],
                         ibufs[b], isem[b])

    def wait_i(b):
        pltpu.make_async_copy(gidx_hbm.at[pl.ds(0, CHUNK)],
                              ibufs[b], isem[b]).wait()

    def start_g(i, b):
        pltpu.async_copy(data_hbm.at[ibufs[b]], rows[b], gsem[b])

    def wait_g(b):
        pltpu.make_async_copy(data_hbm.at[ibufs[b]], rows[b], gsem[b]).wait()

    def start_si(i, b):
        pltpu.async_copy(sidx_hbm.at[pl.ds(gbase + i * CHUNK, CHUNK)],
                         sbufs[b], sisem[b])

    def wait_si(b):
        pltpu.make_async_copy(sidx_hbm.at[pl.ds(0, CHUNK)],
                              sbufs[b], sisem[b]).wait()

    def start_s(i, b):
        wait_si(b)
        pltpu.async_copy(rows[b], acc_sh.at[sbufs[b]], ssem[b], add=True)

    def wait_s(b):
        pltpu.make_async_copy(rows[b], acc_sh.at[sbufs[0]], ssem[b]).wait()

    for b in range(nbuf):
        start_i(b, b)
        start_si(b, b)
    _init_zrow(zrow_v)
    _zero_rows_async(zrow_v, acc_sh, s * rows_t, rows_t // 8, ssem[0])
    plsc.subcore_barrier()

    for b in range(nbuf - 1):
        wait_i(b)
        start_g(b, b)

    def body(k, _):
        for b in range(nbuf):
            i = k * nbuf + b
            wait_g(b)

            @pl.when(i + nbuf < NCH_T)
            def _():
                start_i(i + nbuf, b)

            start_s(i, b)

            @pl.when(i >= 1)
            def _():
                wait_s((b - 1) % nbuf)

                @pl.when(i - 1 + nbuf < NCH_T)
                def _():
                    start_si(i - 1 + nbuf, (b - 1) % nbuf)

            @pl.when(i + nbuf - 1 < NCH_T)
            def _():
                wait_i((b - 1) % nbuf)
                start_g(i + nbuf - 1, (b - 1) % nbuf)
        return 0

    lax.fori_loop(0, NCH_T // nbuf, body, 0)
    for i in range(NCH_T - NCH_T % nbuf, NCH_T):
        b = i % nbuf
        wait_g(b)
        start_s(i, b)
        wait_s((b - 1) % nbuf)
    wait_s((NCH_T - 1) % nbuf)

    plsc.subcore_barrier()
    pltpu.sync_copy(acc_sh.at[pl.ds(s * rows_t, rows_t)],
                    out_hbm.at[c, pl.ds(s * rows_t, rows_t)])


def _make_sc_stage(acc_rows, nbuf):
    return functools.partial(
        pl.kernel,
        out_type=jax.ShapeDtypeStruct((2, acc_rows, NHID), _f32),
        mesh=plsc.VectorSubcoreMesh(core_axis_name="c", subcore_axis_name="s"),
        scratch_types=[
            pltpu.VMEM_SHARED((acc_rows, NHID), _f32),
        ] + [pltpu.VMEM((CHUNK,), _i32) for _ in range(2 * nbuf)]
          + [pltpu.VMEM((CHUNK, NHID), _f32) for _ in range(nbuf)] + [
            pltpu.VMEM((8, NHID), _f32),
        ] + [pltpu.SemaphoreType.DMA for _ in range(4 * nbuf)],
    )(functools.partial(_sc_stage_body, acc_rows, nbuf))


_sc_edge = _make_sc_stage(NEP, 6)
_sc_node = _make_sc_stage(NP, 4)


def _k0_body(x_ref, w_ref, b_ref, o_ref):
    acc = jnp.dot(x_ref[...], w_ref[...], preferred_element_type=_f32)
    o_ref[...] = jnp.maximum(acc + b_ref[...], 0.0)


def _tc_input_layer(x, w0, b0):
    return pl.pallas_call(
        _k0_body,
        grid=(NP // ROW_BLK,),
        in_specs=[
            pl.BlockSpec((ROW_BLK, NHID), lambda i: (i, 0)),
            pl.BlockSpec((NHID, NHID), lambda i: (0, 0)),
            pl.BlockSpec((1, NHID), lambda i: (0, 0)),
        ],
        out_specs=pl.BlockSpec((ROW_BLK, NHID), lambda i: (i, 0)),
        out_shape=jax.ShapeDtypeStruct((NP, NHID), _f32),
    )(x, w0, b0.reshape(1, NHID))


def _kc_body(p_ref, d_ref, o_ref):
    d = d_ref[...]
    o_ref[...] = (p_ref[0] + p_ref[1]) * (d * d * d)


def _tc_edge_scale(xep, dE):
    return pl.pallas_call(
        _kc_body,
        grid=(NEP // ROW_BLK,),
        in_specs=[
            pl.BlockSpec((2, ROW_BLK, NHID), lambda i: (0, i, 0)),
            pl.BlockSpec((ROW_BLK, 1), lambda i: (i, 0)),
        ],
        out_specs=pl.BlockSpec((ROW_BLK, NHID), lambda i: (i, 0)),
        out_shape=jax.ShapeDtypeStruct((NEP, NHID), _f32),
    )(xep, dE)


def _k1_body(beta, xvp_ref, dv_ref, h0_ref, w_ref, o_ref):
    xv = xvp_ref[0] + xvp_ref[1]
    xi = xv * dv_ref[...] + 0.1 * h0_ref[...]
    acc = jnp.dot(xi, w_ref[...], preferred_element_type=_f32)
    o_ref[...] = jnp.maximum((1.0 - beta) * xi + beta * acc, 0.0)


def _tc_gcnii_layer(xvp, dv9, h0, w, beta):
    return pl.pallas_call(
        functools.partial(_k1_body, beta),
        grid=(NP // ROW_BLK,),
        in_specs=[
            pl.BlockSpec((2, ROW_BLK, NHID), lambda i: (0, i, 0)),
            pl.BlockSpec((ROW_BLK, 1), lambda i: (i, 0)),
            pl.BlockSpec((ROW_BLK, NHID), lambda i: (i, 0)),
            pl.BlockSpec((NHID, NHID), lambda i: (0, 0)),
        ],
        out_specs=pl.BlockSpec((ROW_BLK, NHID), lambda i: (i, 0)),
        out_shape=jax.ShapeDtypeStruct((NP, NHID), _f32),
    )(xvp, dv9, h0, w)


def _k1o_body(beta, xvp_ref, dv_ref, h0_ref, w_ref, wo_ref, bo_ref, o_ref):
    xv = xvp_ref[0] + xvp_ref[1]
    xi = xv * dv_ref[...] + 0.1 * h0_ref[...]
    acc = jnp.dot(xi, w_ref[...], preferred_element_type=_f32)
    h = jnp.maximum((1.0 - beta) * xi + beta * acc, 0.0)
    o_ref[...] = jnp.dot(h, wo_ref[...], preferred_element_type=_f32) + bo_ref[...]


def _tc_gcnii_out_layer(xvp, dv9, h0, w, beta, wout, bout):
    return pl.pallas_call(
        functools.partial(_k1o_body, beta),
        grid=(NP // ROW_BLK,),
        in_specs=[
            pl.BlockSpec((2, ROW_BLK, NHID), lambda i: (0, i, 0)),
            pl.BlockSpec((ROW_BLK, 1), lambda i: (i, 0)),
            pl.BlockSpec((ROW_BLK, NHID), lambda i: (i, 0)),
            pl.BlockSpec((NHID, NHID), lambda i: (0, 0)),
            pl.BlockSpec((NHID, NCLASS), lambda i: (0, 0)),
            pl.BlockSpec((1, NCLASS), lambda i: (0, 0)),
        ],
        out_specs=pl.BlockSpec((ROW_BLK, NCLASS), lambda i: (i, 0)),
        out_shape=jax.ShapeDtypeStruct((NP, NCLASS), _f32),
    )(xvp, dv9, h0, w, wout, bout.reshape(1, NCLASS))


def kernel(x, vertex, edges, degE, degV, W0, b0, W1, W2, Wout, bout):
    lamda, alpha = 0.5, 0.1
    edg1 = edges.astype(_i32)

    dE = jnp.pad(degE.reshape(NE, 1), ((0, NEP - NE), (0, 0)))
    dv9 = jnp.pad((1.0 - alpha) * degV, ((0, NP - N), (0, 0)))
    xp = jnp.pad(x, ((0, NP - N), (0, 0)))

    h = _tc_input_layer(xp, W0, b0)
    h0 = h
    betas = [math.log(lamda / (i + 1) + 1) for i in range(2)]

    vtx1 = vertex.astype(_i32)

    vt = vertex.astype(_i32).reshape(CHUNK, NNZ // CHUNK).T
    et = edges.astype(_i32).reshape(CHUNK, NNZ // CHUNK).T
    sv1 = vt.reshape(NNZ)
    se1 = et.reshape(NNZ)

    for i in range(2):
        xep = _sc_edge(h, vtx1, edg1)
        xe = _tc_edge_scale(xep, dE)
        xvp = _sc_node(xe, se1, sv1)
        if i == 0:
            h = _tc_gcnii_layer(xvp, dv9, h0, W1, betas[0])
        else:
            return _tc_gcnii_out_layer(xvp, dv9, h0, W2, betas[1],
                                       Wout, bout)[:N]
